# baseline (speedup 1.0000x reference)
import jax
import jax.numpy as jnp
from jax.experimental import pallas as pl
from jax.experimental.pallas import tpu as pltpu

_BLOCK_ROWS = 4000


def _matmul_block(x_ref, wp_ref, o_ref):
    o_ref[...] = jax.lax.dot_general(
        x_ref[...].astype(jnp.bfloat16),
        wp_ref[...].astype(jnp.bfloat16),
        (((1,), (0,)), ((), ())),
        preferred_element_type=jnp.float32,
    )


def kernel(x, W):
    n, nfeat = x.shape
    nclass = W.shape[0]
    bn = _BLOCK_ROWS
    wp = jnp.zeros((nfeat, nfeat), jnp.float32).at[:, :nclass].set(W.T)
    grid = (pl.cdiv(n, bn),)
    out = pl.pallas_call(
        _matmul_block,
        grid=grid,
        in_specs=[
            pl.BlockSpec((bn, nfeat), lambda i: (i, 0)),
            pl.BlockSpec((nfeat, nfeat), lambda i: (0, 0)),
        ],
        out_specs=pl.BlockSpec((bn, nfeat), lambda i: (i, 0)),
        out_shape=jax.ShapeDtypeStruct((n, nfeat), jnp.float32),
        compiler_params=pltpu.CompilerParams(
            dimension_semantics=("arbitrary",),
        ),
    )(x, wp)
    return out[:, :nclass]


# speedup vs baseline: 1.1761x; 1.1761x over previous
import jax
import jax.numpy as jnp
from jax.experimental import pallas as pl
from jax.experimental.pallas import tpu as pltpu

_BLOCK_ROWS = 4000


def _matmul_block(x_ref, w_ref, o_ref):
    o_ref[...] = jax.lax.dot_general(
        x_ref[...].astype(jnp.bfloat16),
        w_ref[...].astype(jnp.bfloat16),
        (((1,), (1,)), ((), ())),
        preferred_element_type=jnp.float32,
    )


def kernel(x, W):
    n, nfeat = x.shape
    nclass = W.shape[0]
    bn = _BLOCK_ROWS
    grid = (pl.cdiv(n, bn),)
    out = pl.pallas_call(
        _matmul_block,
        grid=grid,
        in_specs=[
            pl.BlockSpec((bn, nfeat), lambda i: (i, 0)),
            pl.BlockSpec((nclass, nfeat), lambda i: (0, 0)),
        ],
        out_specs=pl.BlockSpec((bn, nclass), lambda i: (i, 0)),
        out_shape=jax.ShapeDtypeStruct((n, nclass), jnp.float32),
        compiler_params=pltpu.CompilerParams(
            dimension_semantics=("arbitrary",),
        ),
    )(x, W)
    return out


# speedup vs baseline: 1.3023x; 1.1073x over previous
import jax
import jax.numpy as jnp
from jax.experimental import pallas as pl
from jax.experimental.pallas import tpu as pltpu

_BLOCK_ROWS = 20000


def _matmul_block(x_ref, w_ref, o_ref):
    o_ref[...] = jax.lax.dot_general(
        x_ref[...].astype(jnp.bfloat16),
        w_ref[...].astype(jnp.bfloat16),
        (((1,), (1,)), ((), ())),
        preferred_element_type=jnp.float32,
    )


def kernel(x, W):
    n, nfeat = x.shape
    nclass = W.shape[0]
    bn = _BLOCK_ROWS
    grid = (pl.cdiv(n, bn),)
    out = pl.pallas_call(
        _matmul_block,
        grid=grid,
        in_specs=[
            pl.BlockSpec((bn, nfeat), lambda i: (i, 0)),
            pl.BlockSpec((nclass, nfeat), lambda i: (0, 0)),
        ],
        out_specs=pl.BlockSpec((bn, nclass), lambda i: (i, 0)),
        out_shape=jax.ShapeDtypeStruct((n, nclass), jnp.float32),
        compiler_params=pltpu.CompilerParams(
            dimension_semantics=("arbitrary",),
        ),
    )(x, W)
    return out
